# baseline (speedup 1.0000x reference)
import functools

import jax
import jax.numpy as jnp
from jax import lax
from jax.experimental import pallas as pl
from jax.experimental.pallas import tpu as pltpu
from jax.experimental.pallas import tpu_sc as plsc

WS = 8
L = 16
NC = 2
NS = 16
NW = NC * NS
SUB = 25600
CB = 2 * SUB + 256


def _i32(x):
    return x.astype(jnp.int32)


def _bucket_of(idv, th):
    b = (idv >= th[0]).astype(jnp.int32)
    for k in range(1, 7):
        b = b + (idv >= th[k]).astype(jnp.int32)
    return b


def _load_threshold_splats(thv):
    return [plsc.load_gather(thv, [jnp.full((L,), k + 1, jnp.int32)])
            for k in range(7)]


def _lane(v, b, iota):
    return jnp.sum(jnp.where(iota == jnp.int32(b), v, jnp.int32(0)),
                   dtype=jnp.int32)


def _phase1_body(C, R, ids_hbm, th_hbm, counts_hbm, idbuf, thv, histv):
    wid = lax.axis_index("s") * NC + lax.axis_index("c")
    base = wid * jnp.int32(C)
    pltpu.sync_copy(th_hbm, thv)
    th = _load_threshold_splats(thv)

    def vec_body(i, _):
        idv = idbuf[pl.ds(i * jnp.int32(L), L)]
        b = _bucket_of(idv, th)
        occ, last = plsc.scan_count(b)
        cur = plsc.load_gather(histv, [b])
        plsc.store_scatter(histv, [b], cur + occ, mask=last)
        return 0

    def sub_body(s, _):
        pltpu.sync_copy(ids_hbm.at[pl.ds(base + s * jnp.int32(SUB), SUB)], idbuf)
        histv[...] = jnp.zeros((L,), jnp.int32)
        lax.fori_loop(jnp.int32(0), jnp.int32(SUB // L), vec_body, 0)
        pltpu.sync_copy(
            histv, counts_hbm.at[pl.ds((wid * jnp.int32(R) + s) * jnp.int32(L), L)])
        return 0

    lax.fori_loop(jnp.int32(0), jnp.int32(R), sub_body, 0)


def _phase2_body(C, R, ids_hbm, th_hbm, counts_hbm, inv_hbm, out_hbm, cnt_hbm,
                 idbuf, thv, histv, cntsv, tmpv, deltam, posbuf, cbuf,
                 edgeidx, edgeval, dsem):
    wid = lax.axis_index("s") * NC + lax.axis_index("c")
    base = wid * jnp.int32(C)
    pltpu.sync_copy(th_hbm, thv)
    pltpu.sync_copy(counts_hbm, cntsv)
    th = _load_threshold_splats(thv)
    iota = lax.iota(jnp.int32, L)
    zero = jnp.zeros((L,), jnp.int32)

    def row_body(r, carry):
        tot, pre = carry
        row = cntsv[pl.ds(r * jnp.int32(L), L)]
        return tot + row, pre + jnp.where(r < wid * jnp.int32(R), row, zero)

    tot, pre = lax.fori_loop(jnp.int32(0), jnp.int32(NW * R), row_body,
                             (zero, zero))
    startv = (plsc.cumsum(tot) - tot) + pre

    @pl.when(wid == 0)
    def _():
        tmpv[...] = tot
        pltpu.sync_copy(tmpv, cnt_hbm)

    def z_body(i, _):
        posbuf[pl.ds(i * jnp.int32(L), L)] = zero
        return 0

    lax.fori_loop(jnp.int32(0), jnp.int32(2 * SUB // L), z_body, 0)

    def zc_body(i, _):
        cbuf[pl.ds(i * jnp.int32(L), L)] = zero
        return 0

    lax.fori_loop(jnp.int32(0), jnp.int32(CB // L), zc_body, 0)

    def vec_body(i, _):
        idv = idbuf[pl.ds(i * jnp.int32(L), L)]
        b = _bucket_of(idv, th)
        occ, last = plsc.scan_count(b)
        cur = plsc.load_gather(histv, [b])
        plsc.store_scatter(histv, [b], cur + occ, mask=last)
        pos = cur + occ - 1
        plsc.store_scatter(posbuf, [i * jnp.int32(2 * L) + iota * jnp.int32(2)],
                           pos)
        d = plsc.load_gather(deltam, [b])
        plsc.store_scatter(cbuf, [pos * jnp.int32(2) - d], idv - b * th[0])
        return 0

    def sub_body(s, runv):
        off = base + s * jnp.int32(SUB)
        pltpu.sync_copy(ids_hbm.at[pl.ds(off, SUB)], idbuf)
        cntv = cntsv[pl.ds((wid * jnp.int32(R) + s) * jnp.int32(L), L)]
        g0s = [_lane(runv, b, iota) * jnp.int32(2) for b in range(WS)]
        ecs = [_lane(cntv, b, iota) for b in range(WS)]
        cs = [c * jnp.int32(2) for c in ecs]
        l0s = []
        prev_end = jnp.int32(0)
        for b in range(WS):
            aligned = ((prev_end + jnp.int32(7)) // jnp.int32(8)) * jnp.int32(8)
            lb = aligned + (g0s[b] % jnp.int32(8))
            l0s.append(lb)
            prev_end = lb + cs[b]
        deltav = zero
        for b in range(WS):
            deltav = deltav + jnp.where(iota == jnp.int32(b),
                                        g0s[b] - l0s[b], zero)
        deltam[...] = deltav
        histv[...] = runv
        lax.fori_loop(jnp.int32(0), jnp.int32(SUB // L), vec_body, 0)
        pltpu.sync_copy(posbuf,
                        inv_hbm.at[pl.ds(off * jnp.int32(2), 2 * SUB)])
        for b in range(WS):
            g0, c, l0 = g0s[b], cs[b], l0s[b]

            @pl.when(c > jnp.int32(0))
            def _(g0=g0, c=c, l0=l0):
                head = jnp.minimum((jnp.int32(8) - g0 % jnp.int32(8))
                                   % jnp.int32(8), c)
                tailn = jnp.where(c > head, (c - head) % jnp.int32(8),
                                  jnp.int32(0))
                mid = c - head - tailn
                in_head = iota < head
                in_tail = (iota >= head) & (iota < head + tailn)
                toff = c - tailn - head
                srcv = jnp.where(in_head, l0 + iota,
                                 jnp.where(in_tail, l0 + toff + iota,
                                           jnp.full((L,), 1, jnp.int32) * l0))
                dstv = jnp.where(in_head, g0 + iota,
                                 jnp.where(in_tail, g0 + toff + iota,
                                           jnp.full((L,), 1, jnp.int32) * g0))
                edgeval[...] = plsc.load_gather(cbuf, [srcv])
                edgeidx[...] = dstv
                pltpu.async_copy(edgeval, out_hbm.at[edgeidx], dsem).wait()

                m0 = l0 + head
                d0 = g0 + head
                rem = mid
                cur0 = jnp.int32(0)
                for blk in (1024, 128, 32, 8):
                    nb = (rem // jnp.int32(blk)).astype(jnp.int32)
                    rem = rem % jnp.int32(blk)

                    def t_body(j, _, blk=blk, m0=m0, d0=d0, cur0=cur0):
                        so = pl.multiple_of(m0 + cur0 + j * jnp.int32(blk), 8)
                        do = pl.multiple_of(d0 + cur0 + j * jnp.int32(blk), 8)
                        pltpu.sync_copy(cbuf.at[pl.ds(so, blk)],
                                        out_hbm.at[pl.ds(do, blk)])
                        return 0

                    lax.fori_loop(jnp.int32(0), nb, t_body, 0)
                    cur0 = cur0 + nb * jnp.int32(blk)

        return runv + cntv

    lax.fori_loop(jnp.int32(0), jnp.int32(R), sub_body, startv)


def kernel(ids, block_size):
    n = ids.shape[0]
    assert n % (NW * SUB) == 0
    C = n // NW
    R = C // SUB

    ids32 = ids.astype(jnp.int32)
    bs32 = jnp.asarray(block_size, jnp.int32)
    th = jnp.arange(0, 8, dtype=jnp.int32) * bs32
    th16 = jnp.concatenate([th, jnp.zeros(8, jnp.int32)])

    mesh = plsc.VectorSubcoreMesh(core_axis_name="c", subcore_axis_name="s")

    counts = pl.kernel(
        functools.partial(_phase1_body, C, R),
        out_type=jax.ShapeDtypeStruct((NW * R * L,), jnp.int32),
        mesh=mesh,
        scratch_types=[
            pltpu.VMEM((SUB,), jnp.int32),
            pltpu.VMEM((L,), jnp.int32),
            pltpu.VMEM((L,), jnp.int32),
        ],
        compiler_params=pltpu.CompilerParams(needs_layout_passes=False),
        name="bucketize_phase1_hist",
    )(ids32, th16)

    invp, outp, cnt16 = pl.kernel(
        functools.partial(_phase2_body, C, R),
        out_type=(
            jax.ShapeDtypeStruct((2 * n,), jnp.int32),
            jax.ShapeDtypeStruct((2 * n,), jnp.int32),
            jax.ShapeDtypeStruct((L,), jnp.int32),
        ),
        mesh=mesh,
        scratch_types=[
            pltpu.VMEM((SUB,), jnp.int32),
            pltpu.VMEM((L,), jnp.int32),
            pltpu.VMEM((L,), jnp.int32),
            pltpu.VMEM((NW * R * L,), jnp.int32),
            pltpu.VMEM((L,), jnp.int32),
            pltpu.VMEM((L,), jnp.int32),
            pltpu.VMEM((2 * SUB,), jnp.int32),
            pltpu.VMEM((CB,), jnp.int32),
            pltpu.VMEM((L,), jnp.int32),
            pltpu.VMEM((L,), jnp.int32),
            pltpu.SemaphoreType.DMA,
        ],
        compiler_params=pltpu.CompilerParams(needs_layout_passes=False),
        name="bucketize_phase2_permute",
    )(ids32, th16, counts)

    bucketized_indices = lax.bitcast_convert_type(
        outp.reshape(n, 2), jnp.int64)
    lengths = cnt16[:WS]
    unbucketize_permute = lax.bitcast_convert_type(
        invp.reshape(n, 2), jnp.int64)
    return (bucketized_indices, lengths, unbucketize_permute, lengths)


# speedup vs baseline: 6.4556x; 6.4556x over previous
import functools

import jax
import jax.numpy as jnp
from jax import lax
from jax.experimental import pallas as pl
from jax.experimental.pallas import tpu as pltpu
from jax.experimental.pallas import tpu_sc as plsc

WS = 8
L = 16
NC = 2
NS = 16
NW = NC * NS
SUB = 25600
ROWS = SUB // 128
CB = SUB + 128


def _i32(x):
    return x.astype(jnp.int32)


def _bucket_of(idv, th):
    b = (idv >= th[0]).astype(jnp.int32)
    for k in range(1, 7):
        b = b + (idv >= th[k]).astype(jnp.int32)
    return b


def _load_threshold_splats(thv):
    return [plsc.load_gather(thv, [jnp.full((L,), k + 1, jnp.int32)])
            for k in range(7)]


def _lane(v, b, iota):
    return jnp.sum(jnp.where(iota == jnp.int32(b), v, jnp.int32(0)),
                   dtype=jnp.int32)


def _phase1_body(C, R, ids_hbm, th_hbm, counts_hbm, idbuf, thv, histv):
    wid = lax.axis_index("s") * NC + lax.axis_index("c")
    base = wid * jnp.int32(C)
    pltpu.sync_copy(th_hbm, thv)
    th = _load_threshold_splats(thv)

    def vec_body(i, _):
        for t in range(2):
            idv = idbuf[pl.ds((i * jnp.int32(2) + jnp.int32(t)) * jnp.int32(L),
                              L)]
            b = _bucket_of(idv, th)
            occ, last = plsc.scan_count(b)
            cur = plsc.load_gather(histv, [b])
            plsc.store_scatter(histv, [b], cur + occ, mask=last)
        return 0

    def sub_body(s, _):
        pltpu.sync_copy(ids_hbm.at[pl.ds(base + s * jnp.int32(SUB), SUB)], idbuf)
        histv[...] = jnp.zeros((L,), jnp.int32)
        lax.fori_loop(jnp.int32(0), jnp.int32(SUB // L // 2), vec_body, 0)
        pltpu.sync_copy(
            histv, counts_hbm.at[pl.ds((wid * jnp.int32(R) + s) * jnp.int32(L), L)])
        return 0

    lax.fori_loop(jnp.int32(0), jnp.int32(R), sub_body, 0)


def _phase2_body(C, R, ids_hbm, th_hbm, counts_hbm, inv_hbm, out_hbm, cnt_hbm,
                 idbuf, thv, histv, cntsv, tmpv, deltam, posbuf, cbuf,
                 edgeidx, edgeval, dsem):
    wid = lax.axis_index("s") * NC + lax.axis_index("c")
    base = wid * jnp.int32(C)
    pltpu.sync_copy(th_hbm, thv)
    pltpu.sync_copy(counts_hbm, cntsv)
    th = _load_threshold_splats(thv)
    iota = lax.iota(jnp.int32, L)
    zero = jnp.zeros((L,), jnp.int32)

    def row_body(r, carry):
        tot, pre = carry
        row = cntsv[pl.ds(r * jnp.int32(L), L)]
        return tot + row, pre + jnp.where(r < wid * jnp.int32(R), row, zero)

    tot, pre = lax.fori_loop(jnp.int32(0), jnp.int32(NW * R), row_body,
                             (zero, zero))
    startv = (plsc.cumsum(tot) - tot) + pre

    @pl.when(wid == 0)
    def _():
        tmpv[...] = tot
        pltpu.sync_copy(tmpv, cnt_hbm)

    def vec_body(i, _):
        for t in range(2):
            iv = i * jnp.int32(2) + jnp.int32(t)
            idv = idbuf[pl.ds(iv * jnp.int32(L), L)]
            b = _bucket_of(idv, th)
            occ, last = plsc.scan_count(b)
            cur = plsc.load_gather(histv, [b])
            plsc.store_scatter(histv, [b], cur + occ, mask=last)
            pos = cur + occ - 1
            r = iv // jnp.int32(8)
            col = (iv % jnp.int32(8)) * jnp.int32(L)
            posbuf[r, pl.ds(col, L)] = pos
            d = plsc.load_gather(deltam, [b])
            plsc.store_scatter(cbuf, [pos - d], idv - b * th[0])
        return 0

    def sub_body(s, runv):
        off = base + s * jnp.int32(SUB)
        pltpu.sync_copy(ids_hbm.at[pl.ds(off, SUB)], idbuf)
        cntv = cntsv[pl.ds((wid * jnp.int32(R) + s) * jnp.int32(L), L)]
        g0s = [_lane(runv, b, iota) for b in range(WS)]
        cs = [_lane(cntv, b, iota) for b in range(WS)]
        l0s = []
        prev_end = jnp.int32(0)
        for b in range(WS):
            aligned = ((prev_end + jnp.int32(7)) // jnp.int32(8)) * jnp.int32(8)
            lb = aligned + (g0s[b] % jnp.int32(8))
            l0s.append(lb)
            prev_end = lb + cs[b]
        deltav = zero
        for b in range(WS):
            deltav = deltav + jnp.where(iota == jnp.int32(b),
                                        g0s[b] - l0s[b], zero)
        deltam[...] = deltav
        histv[...] = runv
        lax.fori_loop(jnp.int32(0), jnp.int32(SUB // L // 2), vec_body, 0)
        row_off = pl.multiple_of(off // jnp.int32(128), 8)
        pltpu.sync_copy(posbuf, inv_hbm.at[pl.ds(row_off, ROWS)])
        for b in range(WS):
            g0, c, l0 = g0s[b], cs[b], l0s[b]

            @pl.when(c > jnp.int32(0))
            def _(g0=g0, c=c, l0=l0):
                head = jnp.minimum((jnp.int32(8) - g0 % jnp.int32(8))
                                   % jnp.int32(8), c)
                tailn = jnp.where(c > head, (c - head) % jnp.int32(8),
                                  jnp.int32(0))
                mid = c - head - tailn
                in_head = iota < head
                in_tail = (iota >= head) & (iota < head + tailn)
                toff = c - tailn - head
                srcv = jnp.where(in_head, l0 + iota,
                                 jnp.where(in_tail, l0 + toff + iota,
                                           jnp.full((L,), 1, jnp.int32) * l0))
                dstv = jnp.where(in_head, g0 + iota,
                                 jnp.where(in_tail, g0 + toff + iota,
                                           jnp.full((L,), 1, jnp.int32) * g0))
                edgeval[...] = plsc.load_gather(cbuf, [srcv])
                edgeidx[...] = dstv
                pltpu.async_copy(edgeval, out_hbm.at[edgeidx], dsem).wait()

                m0 = l0 + head
                d0 = g0 + head
                rem = mid
                cur0 = jnp.int32(0)
                for blk in (1024, 128, 32, 8):
                    nb = (rem // jnp.int32(blk)).astype(jnp.int32)
                    rem = rem % jnp.int32(blk)

                    def t_body(j, _, blk=blk, m0=m0, d0=d0, cur0=cur0):
                        so = pl.multiple_of(m0 + cur0 + j * jnp.int32(blk), 8)
                        do = pl.multiple_of(d0 + cur0 + j * jnp.int32(blk), 8)
                        pltpu.sync_copy(cbuf.at[pl.ds(so, blk)],
                                        out_hbm.at[pl.ds(do, blk)])
                        return 0

                    lax.fori_loop(jnp.int32(0), nb, t_body, 0)
                    cur0 = cur0 + nb * jnp.int32(blk)

        return runv + cntv

    lax.fori_loop(jnp.int32(0), jnp.int32(R), sub_body, startv)


def kernel(ids, block_size):
    n = ids.shape[0]
    assert n % (NW * SUB) == 0
    C = n // NW
    R = C // SUB

    ids32 = ids.astype(jnp.int32)
    bs32 = jnp.asarray(block_size, jnp.int32)
    th = jnp.arange(0, 8, dtype=jnp.int32) * bs32
    th16 = jnp.concatenate([th, jnp.zeros(8, jnp.int32)])

    mesh = plsc.VectorSubcoreMesh(core_axis_name="c", subcore_axis_name="s")

    counts = pl.kernel(
        functools.partial(_phase1_body, C, R),
        out_type=jax.ShapeDtypeStruct((NW * R * L,), jnp.int32),
        mesh=mesh,
        scratch_types=[
            pltpu.VMEM((SUB,), jnp.int32),
            pltpu.VMEM((L,), jnp.int32),
            pltpu.VMEM((L,), jnp.int32),
        ],
        compiler_params=pltpu.CompilerParams(needs_layout_passes=False),
        name="bucketize_phase1_hist",
    )(ids32, th16)

    inv2d, out_vals, cnt16 = pl.kernel(
        functools.partial(_phase2_body, C, R),
        out_type=(
            jax.ShapeDtypeStruct((n // 128, 128), jnp.int32),
            jax.ShapeDtypeStruct((n,), jnp.int32),
            jax.ShapeDtypeStruct((L,), jnp.int32),
        ),
        mesh=mesh,
        scratch_types=[
            pltpu.VMEM((SUB,), jnp.int32),
            pltpu.VMEM((L,), jnp.int32),
            pltpu.VMEM((L,), jnp.int32),
            pltpu.VMEM((NW * R * L,), jnp.int32),
            pltpu.VMEM((L,), jnp.int32),
            pltpu.VMEM((L,), jnp.int32),
            pltpu.VMEM((ROWS, 128), jnp.int32),
            pltpu.VMEM((CB,), jnp.int32),
            pltpu.VMEM((L,), jnp.int32),
            pltpu.VMEM((L,), jnp.int32),
            pltpu.SemaphoreType.DMA,
        ],
        compiler_params=pltpu.CompilerParams(needs_layout_passes=False),
        name="bucketize_phase2_permute",
    )(ids32, th16, counts)

    bucketized_indices = out_vals.astype(jnp.int64)
    lengths = cnt16[:WS]
    unbucketize_permute = inv2d.reshape(n).astype(jnp.int64)
    return (bucketized_indices, lengths, unbucketize_permute, lengths)


# speedup vs baseline: 6.4653x; 1.0015x over previous
import functools

import jax
import jax.numpy as jnp
from jax import lax
from jax.experimental import pallas as pl
from jax.experimental.pallas import tpu as pltpu
from jax.experimental.pallas import tpu_sc as plsc

WS = 8
L = 16
NC = 2
NS = 16
NW = NC * NS
SUB = 25600
ROWS = SUB // 128
CB = SUB + 128


def _bucket_of(idv, th):
    b = (idv >= th[0]).astype(jnp.int32)
    for k in range(1, 7):
        b = b + (idv >= th[k]).astype(jnp.int32)
    return b


def _load_threshold_splats(thv):
    return [plsc.load_gather(thv, [jnp.full((L,), k + 1, jnp.int32)])
            for k in range(7)]


def _lane(v, b, iota):
    return jnp.sum(jnp.where(iota == jnp.int32(b), v, jnp.int32(0)),
                   dtype=jnp.int32)


def _phase1_body(C, R, ids_hbm, th_hbm, counts_hbm, idbuf, thv, histv):
    wid = lax.axis_index("s") * NC + lax.axis_index("c")
    base = wid * jnp.int32(C)
    pltpu.sync_copy(th_hbm, thv)
    th = _load_threshold_splats(thv)

    def vec_body(i, _):
        for t in range(2):
            idv = idbuf[pl.ds((i * jnp.int32(2) + jnp.int32(t)) * jnp.int32(L),
                              L)]
            b = _bucket_of(idv, th)
            occ, last = plsc.scan_count(b)
            cur = plsc.load_gather(histv, [b])
            plsc.store_scatter(histv, [b], cur + occ, mask=last)
        return 0

    def sub_body(s, _):
        pltpu.sync_copy(ids_hbm.at[pl.ds(base + s * jnp.int32(SUB), SUB)], idbuf)
        histv[...] = jnp.zeros((L,), jnp.int32)
        lax.fori_loop(jnp.int32(0), jnp.int32(SUB // L // 2), vec_body, 0)
        pltpu.sync_copy(
            histv, counts_hbm.at[pl.ds((wid * jnp.int32(R) + s) * jnp.int32(L), L)])
        return 0

    lax.fori_loop(jnp.int32(0), jnp.int32(R), sub_body, 0)


def _phase2_body(C, R, ids_hbm, th_hbm, counts_hbm, inv_hbm, out_hbm, cnt_hbm,
                 idbuf, thv, histv, cntsv, tmpv, deltam, posbuf, cbuf,
                 edgeidx, edgeval, dsem):
    wid = lax.axis_index("s") * NC + lax.axis_index("c")
    base = wid * jnp.int32(C)
    pltpu.sync_copy(th_hbm, thv)
    pltpu.sync_copy(counts_hbm, cntsv)
    th = _load_threshold_splats(thv)
    iota = lax.iota(jnp.int32, L)
    zero = jnp.zeros((L,), jnp.int32)

    def row_body(r, carry):
        tot, pre = carry
        row = cntsv[pl.ds(r * jnp.int32(L), L)]
        return tot + row, pre + jnp.where(r < wid * jnp.int32(R), row, zero)

    tot, pre = lax.fori_loop(jnp.int32(0), jnp.int32(NW * R), row_body,
                             (zero, zero))
    startv = (plsc.cumsum(tot) - tot) + pre

    @pl.when(wid == 0)
    def _():
        tmpv[...] = tot
        pltpu.sync_copy(tmpv, cnt_hbm)

    def vec_body(i, _):
        for t in range(2):
            iv = i * jnp.int32(2) + jnp.int32(t)
            idv = idbuf[pl.ds(iv * jnp.int32(L), L)]
            b = _bucket_of(idv, th)
            occ, last = plsc.scan_count(b)
            cur = plsc.load_gather(histv, [b])
            plsc.store_scatter(histv, [b], cur + occ, mask=last)
            pos = cur + occ - 1
            r = iv // jnp.int32(8)
            col = (iv % jnp.int32(8)) * jnp.int32(L)
            posbuf[r, pl.ds(col, L)] = pos
            d = plsc.load_gather(deltam, [b])
            plsc.store_scatter(cbuf, [pos - d], idv - b * th[0])
        return 0

    def sub_body(s, runv):
        off = base + s * jnp.int32(SUB)
        pltpu.sync_copy(ids_hbm.at[pl.ds(off, SUB)], idbuf)
        cntv = cntsv[pl.ds((wid * jnp.int32(R) + s) * jnp.int32(L), L)]
        g0s = [_lane(runv, b, iota) for b in range(WS)]
        cs = [_lane(cntv, b, iota) for b in range(WS)]
        l0s = []
        prev_end = jnp.int32(0)
        for b in range(WS):
            aligned = ((prev_end + jnp.int32(7)) // jnp.int32(8)) * jnp.int32(8)
            lb = aligned + (g0s[b] % jnp.int32(8))
            l0s.append(lb)
            prev_end = lb + cs[b]
        deltav = zero
        for b in range(WS):
            deltav = deltav + jnp.where(iota == jnp.int32(b),
                                        g0s[b] - l0s[b], zero)
        deltam[...] = deltav
        histv[...] = runv
        lax.fori_loop(jnp.int32(0), jnp.int32(SUB // L // 2), vec_body, 0)
        row_off = pl.multiple_of(off // jnp.int32(128), 8)
        pltpu.sync_copy(posbuf, inv_hbm.at[pl.ds(row_off, ROWS)])
        for b in range(WS):
            g0, c, l0 = g0s[b], cs[b], l0s[b]

            @pl.when(c > jnp.int32(0))
            def _(g0=g0, c=c, l0=l0):
                head = jnp.minimum((jnp.int32(8) - g0 % jnp.int32(8))
                                   % jnp.int32(8), c)
                tailn = jnp.where(c > head, (c - head) % jnp.int32(8),
                                  jnp.int32(0))
                mid = c - head - tailn
                in_head = iota < head
                in_tail = (iota >= head) & (iota < head + tailn)
                toff = c - tailn - head
                srcv = jnp.where(in_head, l0 + iota,
                                 jnp.where(in_tail, l0 + toff + iota,
                                           jnp.full((L,), 1, jnp.int32) * l0))
                dstv = jnp.where(in_head, g0 + iota,
                                 jnp.where(in_tail, g0 + toff + iota,
                                           jnp.full((L,), 1, jnp.int32) * g0))
                edgeval[...] = plsc.load_gather(cbuf, [srcv])
                edgeidx[...] = dstv
                pltpu.async_copy(edgeval, out_hbm.at[edgeidx], dsem).wait()

                m0 = l0 + head
                d0 = g0 + head
                rem = mid
                cur0 = jnp.int32(0)
                for blk in (1024, 128, 32, 8):
                    nb = (rem // jnp.int32(blk)).astype(jnp.int32)
                    rem = rem % jnp.int32(blk)

                    def t_body(j, _, blk=blk, m0=m0, d0=d0, cur0=cur0):
                        so = pl.multiple_of(m0 + cur0 + j * jnp.int32(blk), 8)
                        do = pl.multiple_of(d0 + cur0 + j * jnp.int32(blk), 8)
                        pltpu.sync_copy(cbuf.at[pl.ds(so, blk)],
                                        out_hbm.at[pl.ds(do, blk)])
                        return 0

                    lax.fori_loop(jnp.int32(0), nb, t_body, 0)
                    cur0 = cur0 + nb * jnp.int32(blk)

        return runv + cntv

    lax.fori_loop(jnp.int32(0), jnp.int32(R), sub_body, startv)


def kernel(ids, block_size):
    n = ids.shape[0]
    assert n % (NW * SUB) == 0
    C = n // NW
    R = C // SUB

    ids32 = ids.astype(jnp.int32)
    bs32 = jnp.asarray(block_size, jnp.int32)
    th = jnp.arange(0, 8, dtype=jnp.int32) * bs32
    th16 = jnp.concatenate([th, jnp.zeros(8, jnp.int32)])

    mesh = plsc.VectorSubcoreMesh(core_axis_name="c", subcore_axis_name="s")

    counts = pl.kernel(
        functools.partial(_phase1_body, C, R),
        out_type=jax.ShapeDtypeStruct((NW * R * L,), jnp.int32),
        mesh=mesh,
        scratch_types=[
            pltpu.VMEM((SUB,), jnp.int32),
            pltpu.VMEM((L,), jnp.int32),
            pltpu.VMEM((L,), jnp.int32),
        ],
        compiler_params=pltpu.CompilerParams(needs_layout_passes=False),
        name="bucketize_phase1_hist",
    )(ids32, th16)

    inv2d, out_vals, cnt16 = pl.kernel(
        functools.partial(_phase2_body, C, R),
        out_type=(
            jax.ShapeDtypeStruct((n // 128, 128), jnp.int32),
            jax.ShapeDtypeStruct((n,), jnp.int32),
            jax.ShapeDtypeStruct((L,), jnp.int32),
        ),
        mesh=mesh,
        scratch_types=[
            pltpu.VMEM((SUB,), jnp.int32),
            pltpu.VMEM((L,), jnp.int32),
            pltpu.VMEM((L,), jnp.int32),
            pltpu.VMEM((NW * R * L,), jnp.int32),
            pltpu.VMEM((L,), jnp.int32),
            pltpu.VMEM((L,), jnp.int32),
            pltpu.VMEM((ROWS, 128), jnp.int32),
            pltpu.VMEM((CB,), jnp.int32),
            pltpu.VMEM((L,), jnp.int32),
            pltpu.VMEM((L,), jnp.int32),
            pltpu.SemaphoreType.DMA,
        ],
        compiler_params=pltpu.CompilerParams(needs_layout_passes=False),
        name="bucketize_phase2_permute",
    )(ids32, th16, counts)

    bucketized_indices = out_vals.astype(jnp.int64)
    lengths = cnt16[:WS]
    unbucketize_permute = inv2d.reshape(n).astype(jnp.int64)
    return (bucketized_indices, lengths, unbucketize_permute, lengths)
